# baseline (speedup 1.0000x reference)
import jax
import jax.numpy as jnp
from jax.experimental import pallas as pl

_BLOCK = 4000


def _mlp_kernel(x_ref, w_ref, b_ref, o_ref):
    acc = jnp.dot(x_ref[...], w_ref[...], preferred_element_type=jnp.float32)
    o_ref[...] = jnp.maximum(acc + b_ref[...], 0.0)


def kernel(movie_x, user_emb_weight, W, b):
    n, f = movie_x.shape
    e = W.shape[1]
    movie = pl.pallas_call(
        _mlp_kernel,
        grid=(n // _BLOCK,),
        in_specs=[
            pl.BlockSpec((_BLOCK, f), lambda i: (i, 0)),
            pl.BlockSpec((f, e), lambda i: (0, 0)),
            pl.BlockSpec((1, e), lambda i: (0, 0)),
        ],
        out_specs=pl.BlockSpec((_BLOCK, e), lambda i: (i, 0)),
        out_shape=jax.ShapeDtypeStruct((n, e), jnp.float32),
    )(movie_x, W, b.reshape(1, -1))
    return (user_emb_weight, movie)


# speedup vs baseline: 1.0369x; 1.0369x over previous
import jax
import jax.numpy as jnp
from jax.experimental import pallas as pl

_BLOCK = 10000


def _mlp_kernel(x_ref, w_ref, b_ref, o_ref):
    acc = jnp.dot(x_ref[...], w_ref[...], preferred_element_type=jnp.float32)
    o_ref[...] = jnp.maximum(acc + b_ref[...], 0.0)


def kernel(movie_x, user_emb_weight, W, b):
    n, f = movie_x.shape
    e = W.shape[1]
    movie = pl.pallas_call(
        _mlp_kernel,
        grid=(n // _BLOCK,),
        in_specs=[
            pl.BlockSpec((_BLOCK, f), lambda i: (i, 0)),
            pl.BlockSpec((f, e), lambda i: (0, 0)),
            pl.BlockSpec((1, e), lambda i: (0, 0)),
        ],
        out_specs=pl.BlockSpec((_BLOCK, e), lambda i: (i, 0)),
        out_shape=jax.ShapeDtypeStruct((n, e), jnp.float32),
    )(movie_x, W, b.reshape(1, -1))
    return (user_emb_weight, movie)
